# baseline (speedup 1.0000x reference)
import jax
import jax.numpy as jnp
from jax.experimental import pallas as pl
from jax.experimental.pallas import tpu as pltpu

_NF = 128
_NP = 100000
_NCQ = 5000
_SCALE = 30.0
_B = 256
_BLK = 2000
_NSTEPS = _NP // _BLK


def _loss_kernel(inputs_ref, label_ref, lut_ref, cq_ref, out_ref,
                 x_ref, s_ref, t_ref):
    i = pl.program_id(0)
    ones = jnp.ones((1, _NF), dtype=jnp.float32)

    @pl.when(i == 0)
    def _init():
        xin = inputs_ref[:]
        nrm = jnp.sqrt(jnp.sum(xin * xin, axis=1, keepdims=True))
        x = xin / jnp.maximum(nrm, 1e-12)
        x_ref[:] = x
        cqb = cq_ref[:]
        lu = jax.lax.dot_general(x.astype(jnp.bfloat16),
                                 cqb.astype(jnp.bfloat16),
                                 (((1,), (1,)), ((), ())),
                                 preferred_element_type=jnp.float32)
        absum = jax.lax.dot_general(ones, jnp.abs(cqb),
                                    (((1,), (1,)), ((), ())),
                                    preferred_element_type=jnp.float32)
        lu = jnp.where(absum == 0.0, -1.0, lu)
        s_ref[:] = jnp.sum(jnp.exp(_SCALE * lu - _SCALE), axis=1,
                           keepdims=True)
        t_ref[:] = jnp.zeros_like(t_ref)

    x = x_ref[:]
    blk = lut_ref[:]
    logits = jax.lax.dot_general(x.astype(jnp.bfloat16),
                                 blk.astype(jnp.bfloat16),
                                 (((1,), (1,)), ((), ())),
                                 preferred_element_type=jnp.float32)
    absum = jax.lax.dot_general(ones, jnp.abs(blk), (((1,), (1,)), ((), ())),
                                preferred_element_type=jnp.float32)
    bad = absum == 0.0
    l = jnp.where(bad, -1.0, logits)
    lbl = label_ref[:]
    local = lbl - i * _BLK
    in_blk = (local >= 0) & (local < _BLK) & (lbl < _NP)
    cols = jax.lax.broadcasted_iota(jnp.int32, (_B, _BLK), 1)
    is_tgt = in_blk & (cols == local)
    l = jnp.where(is_tgt & bad, 1.0, l)
    s_ref[:] += jnp.sum(jnp.exp(_SCALE * l - _SCALE), axis=1, keepdims=True)
    t_ref[:] += jnp.sum(jnp.where(is_tgt, l, 0.0), axis=1, keepdims=True)

    @pl.when(i == _NSTEPS - 1)
    def _fin():
        lse = _SCALE + jnp.log(s_ref[:])
        per = lse - _SCALE * t_ref[:]
        per = jnp.where(label_ref[:] == _NP, 0.0, per)
        out_ref[:, :] = jnp.sum(per, axis=0, keepdims=True) / _B


def kernel(inputs, label, ious, lut, cq):
    del ious
    lbl2 = label.reshape(_B, 1)
    out = pl.pallas_call(
        _loss_kernel,
        grid=(_NSTEPS,),
        in_specs=[
            pl.BlockSpec((_B, _NF), lambda i: (0, 0)),
            pl.BlockSpec((_B, 1), lambda i: (0, 0)),
            pl.BlockSpec((_BLK, _NF), lambda i: (i, 0)),
            pl.BlockSpec((_NCQ, _NF), lambda i: (0, 0)),
        ],
        out_specs=pl.BlockSpec((1, 1), lambda i: (0, 0)),
        out_shape=jax.ShapeDtypeStruct((1, 1), jnp.float32),
        scratch_shapes=[
            pltpu.VMEM((_B, _NF), jnp.float32),
            pltpu.VMEM((_B, 1), jnp.float32),
            pltpu.VMEM((_B, 1), jnp.float32),
        ],
        compiler_params=pltpu.CompilerParams(
            dimension_semantics=("arbitrary",)),
    )(inputs, lbl2, lut, cq)
    return out[0, 0]


# speedup vs baseline: 1.1835x; 1.1835x over previous
import jax
import jax.numpy as jnp
from jax.experimental import pallas as pl
from jax.experimental.pallas import tpu as pltpu

_NF = 128
_NP = 100000
_NCQ = 5000
_SCALE = 30.0
_B = 256
_BLK = 2000
_NSTEPS = _NP // _BLK


def _loss_kernel(inputs_ref, label_ref, lut_ref, cq_ref, out_ref,
                 x_ref, s_ref, t_ref, nb_ref):
    i = pl.program_id(0)
    ones = jnp.ones((1, _NF), dtype=jnp.bfloat16)

    @pl.when(i == 0)
    def _init():
        xin = inputs_ref[:]
        nrm = jnp.sqrt(jnp.sum(xin * xin, axis=1, keepdims=True))
        x = _SCALE * xin / jnp.maximum(nrm, 1e-12)
        x_ref[:] = x.astype(jnp.bfloat16)
        cqb = cq_ref[:].astype(jnp.bfloat16)
        lu = jax.lax.dot_general(x_ref[:], cqb, (((1,), (1,)), ((), ())),
                                 preferred_element_type=jnp.float32)
        absum = jax.lax.dot_general(ones, jnp.abs(cqb),
                                    (((1,), (1,)), ((), ())),
                                    preferred_element_type=jnp.float32)
        s_ref[:] = jnp.sum(jnp.exp(lu), axis=1, keepdims=True)
        nb_ref[:, :] = jnp.sum((absum == 0.0).astype(jnp.float32), axis=1,
                               keepdims=True)
        t_ref[:] = jnp.zeros_like(t_ref)

    x = x_ref[:]
    blk = lut_ref[:].astype(jnp.bfloat16)
    logits = jax.lax.dot_general(x, blk, (((1,), (1,)), ((), ())),
                                 preferred_element_type=jnp.float32)
    absum = jax.lax.dot_general(ones, jnp.abs(blk), (((1,), (1,)), ((), ())),
                                preferred_element_type=jnp.float32)
    bad = absum == 0.0
    s_ref[:] += jnp.sum(jnp.exp(logits), axis=1, keepdims=True)
    nb_ref[:, :] += jnp.sum(bad.astype(jnp.float32), axis=1, keepdims=True)
    lbl = label_ref[:]
    local = lbl - i * _BLK
    in_blk = (local >= 0) & (local < _BLK) & (lbl < _NP)
    cols = jax.lax.broadcasted_iota(jnp.int32, (_B, _BLK), 1)
    is_tgt = in_blk & (cols == local)
    t_ref[:] += jnp.sum(jnp.where(is_tgt, logits, 0.0), axis=1,
                        keepdims=True)
    bp_f = jnp.sum(jnp.where(is_tgt & bad, 1.0, 0.0), axis=1, keepdims=True)
    s_ref[:] += bp_f * (jnp.exp(_SCALE) - jnp.exp(-_SCALE))
    t_ref[:] += bp_f * _SCALE

    @pl.when(i == _NSTEPS - 1)
    def _fin():
        s = s_ref[:] + nb_ref[:, :] * (jnp.exp(-_SCALE) - 1.0)
        per = jnp.log(s) - t_ref[:]
        per = jnp.where(label_ref[:] == _NP, 0.0, per)
        out_ref[:, :] = jnp.sum(per, axis=0, keepdims=True) / _B


def kernel(inputs, label, ious, lut, cq):
    del ious
    lbl2 = label.reshape(_B, 1)
    out = pl.pallas_call(
        _loss_kernel,
        grid=(_NSTEPS,),
        in_specs=[
            pl.BlockSpec((_B, _NF), lambda i: (0, 0)),
            pl.BlockSpec((_B, 1), lambda i: (0, 0)),
            pl.BlockSpec((_BLK, _NF), lambda i: (i, 0)),
            pl.BlockSpec((_NCQ, _NF), lambda i: (0, 0)),
        ],
        out_specs=pl.BlockSpec((1, 1), lambda i: (0, 0)),
        out_shape=jax.ShapeDtypeStruct((1, 1), jnp.float32),
        scratch_shapes=[
            pltpu.VMEM((_B, _NF), jnp.bfloat16),
            pltpu.VMEM((_B, 1), jnp.float32),
            pltpu.VMEM((_B, 1), jnp.float32),
            pltpu.VMEM((1, 1), jnp.float32),
        ],
        compiler_params=pltpu.CompilerParams(
            dimension_semantics=("arbitrary",)),
    )(inputs, lbl2, lut, cq)
    return out[0, 0]
